# baseline (speedup 1.0000x reference)
import functools

import jax
import jax.numpy as jnp
from jax import lax
from jax.experimental import pallas as pl
from jax.experimental.pallas import tpu as pltpu
from jax.experimental.pallas import tpu_sc as plsc

D_MODEL = 64
SCALE = 8.0

NUM_CORES = 2
NUM_SUBCORES = 16
NUM_WORKERS = NUM_CORES * NUM_SUBCORES

B_TOTAL = 4096 * 200
ROWS_PER_WORKER = B_TOTAL // NUM_WORKERS
CHUNK = 800
NUM_CHUNKS = ROWS_PER_WORKER // CHUNK
LANES = 16


def _emb_body(x_hbm, lut_hbm, out_hbm, idx_v, rows_v, sem):
    wid = lax.axis_index("s") * NUM_CORES + lax.axis_index("c")
    base = wid * ROWS_PER_WORKER

    def chunk_body(g, carry):
        off = base + g * CHUNK
        pltpu.sync_copy(x_hbm.at[pl.ds(off, CHUNK)], idx_v)
        pltpu.async_copy(lut_hbm.at[idx_v], rows_v, sem).wait()

        def row_body(r, c):
            for j in range(D_MODEL // LANES):
                sl = pl.ds(j * LANES, LANES)
                rows_v[r, sl] = rows_v[r, sl] * SCALE
            return c

        lax.fori_loop(0, CHUNK, row_body, 0)
        pltpu.sync_copy(rows_v, out_hbm.at[pl.ds(off, CHUNK)])
        return carry

    lax.fori_loop(0, NUM_CHUNKS, chunk_body, 0)


_emb = functools.partial(
    pl.kernel,
    out_type=jax.ShapeDtypeStruct((B_TOTAL, D_MODEL), jnp.float32),
    mesh=plsc.VectorSubcoreMesh(
        core_axis_name="c",
        subcore_axis_name="s",
        num_cores=NUM_CORES,
        num_subcores=NUM_SUBCORES,
    ),
    scratch_types=[
        pltpu.VMEM((CHUNK,), jnp.int32),
        pltpu.VMEM((CHUNK, D_MODEL), jnp.float32),
        pltpu.SemaphoreType.DMA,
    ],
    compiler_params=pltpu.CompilerParams(use_tc_tiling_on_sc=False),
)(_emb_body)


@jax.jit
def kernel(x, lut):
    flat = _emb(x.reshape(-1), lut)
    return flat.reshape(x.shape + (D_MODEL,))


# speedup vs baseline: 1.1191x; 1.1191x over previous
import functools

import jax
import jax.numpy as jnp
from jax import lax
from jax.experimental import pallas as pl
from jax.experimental.pallas import tpu as pltpu
from jax.experimental.pallas import tpu_sc as plsc

D_MODEL = 64
SCALE = 8.0

NUM_CORES = 2
NUM_SUBCORES = 16
NUM_WORKERS = NUM_CORES * NUM_SUBCORES

B_TOTAL = 4096 * 200
ROWS_PER_WORKER = B_TOTAL // NUM_WORKERS
CHUNK = 400
NUM_CHUNKS = ROWS_PER_WORKER // CHUNK
NBUF = 2
LANES = 16


def _scale_chunk(src, dst):
    @plsc.parallel_loop(0, CHUNK, step=1, unroll=8)
    def _(r):
        for j in range(D_MODEL // LANES):
            sl = pl.ds(j * LANES, LANES)
            dst[r, sl] = src[r, sl] * SCALE


def _emb_body(x_hbm, lut_hbm, out_hbm, idx_v,
              rows0, rows1, stage0, stage1, gs0, gs1, os0, os1):
    wid = lax.axis_index("s") * NUM_CORES + lax.axis_index("c")
    base = wid * ROWS_PER_WORKER
    rows = (rows0, rows1)
    stage = (stage0, stage1)
    gsem = (gs0, gs1)
    osem = (os0, os1)

    pltpu.sync_copy(x_hbm.at[pl.ds(base, ROWS_PER_WORKER)], idx_v)

    def start_gather(g, b):
        pltpu.async_copy(
            lut_hbm.at[idx_v.at[pl.ds(g * CHUNK, CHUNK)]], rows[b], gsem[b])

    def wait_gather(g, b):
        pltpu.make_async_copy(
            lut_hbm.at[idx_v.at[pl.ds(g * CHUNK, CHUNK)]], rows[b],
            gsem[b]).wait()

    def start_out(g, b):
        pltpu.async_copy(
            stage[b], out_hbm.at[pl.ds(base + g * CHUNK, CHUNK)], osem[b])

    def wait_out(g, b):
        pltpu.make_async_copy(
            stage[b], out_hbm.at[pl.ds(base + g * CHUNK, CHUNK)],
            osem[b]).wait()

    start_gather(0, 0)
    start_gather(1, 1)

    @pl.loop(0, NUM_CHUNKS, step=NBUF)
    def _(g):
        for b in range(NBUF):
            gg = g + b
            wait_gather(gg, b)

            @pl.when(gg >= NBUF)
            def _():
                wait_out(gg - NBUF, b)

            _scale_chunk(rows[b], stage[b])

            @pl.when(gg + NBUF < NUM_CHUNKS)
            def _():
                start_gather(gg + NBUF, b)

            start_out(gg, b)

    for b in range(NBUF):
        wait_out(NUM_CHUNKS - NBUF + b, b)


_emb = functools.partial(
    pl.kernel,
    out_type=jax.ShapeDtypeStruct((B_TOTAL, D_MODEL), jnp.float32),
    mesh=plsc.VectorSubcoreMesh(
        core_axis_name="c",
        subcore_axis_name="s",
        num_cores=NUM_CORES,
        num_subcores=NUM_SUBCORES,
    ),
    scratch_types=[
        pltpu.VMEM((ROWS_PER_WORKER,), jnp.int32),
        pltpu.VMEM((CHUNK, D_MODEL), jnp.float32),
        pltpu.VMEM((CHUNK, D_MODEL), jnp.float32),
        pltpu.VMEM((CHUNK, D_MODEL), jnp.float32),
        pltpu.VMEM((CHUNK, D_MODEL), jnp.float32),
        pltpu.SemaphoreType.DMA,
        pltpu.SemaphoreType.DMA,
        pltpu.SemaphoreType.DMA,
        pltpu.SemaphoreType.DMA,
    ],
    compiler_params=pltpu.CompilerParams(use_tc_tiling_on_sc=False),
)(_emb_body)


@jax.jit
def kernel(x, lut):
    flat = _emb(x.reshape(-1), lut)
    return flat.reshape(x.shape + (D_MODEL,))


# speedup vs baseline: 1.3628x; 1.2177x over previous
import functools

import jax
import jax.numpy as jnp
from jax import lax
from jax.experimental import pallas as pl
from jax.experimental.pallas import tpu as pltpu
from jax.experimental.pallas import tpu_sc as plsc

D_MODEL = 64
D_PAD = 128
SCALE = 8.0

NUM_CORES = 2
NUM_SUBCORES = 16
NUM_WORKERS = NUM_CORES * NUM_SUBCORES

B_TOTAL = 4096 * 200
ROWS_PER_WORKER = B_TOTAL // NUM_WORKERS
CHUNK = 200
NUM_CHUNKS = ROWS_PER_WORKER // CHUNK
NBUF = 2
LANES = 16


def _scale_chunk(src, dst):
    @plsc.parallel_loop(0, CHUNK, step=1, unroll=8)
    def _(r):
        for j in range(D_MODEL // LANES):
            sl = pl.ds(j * LANES, LANES)
            dst[r, sl] = src[r, sl] * SCALE


def _emb_body(x_hbm, lut_hbm, out_hbm, idx_v,
              rows0, rows1, stage0, stage1, gs0, gs1, os0, os1):
    wid = lax.axis_index("s") * NUM_CORES + lax.axis_index("c")
    base = wid * ROWS_PER_WORKER
    rows = (rows0, rows1)
    stage = (stage0, stage1)
    gsem = (gs0, gs1)
    osem = (os0, os1)

    pltpu.sync_copy(x_hbm.at[pl.ds(base, ROWS_PER_WORKER)], idx_v)

    def start_gather(g, b):
        pltpu.async_copy(
            lut_hbm.at[idx_v.at[pl.ds(g * CHUNK, CHUNK)]], rows[b], gsem[b])

    def wait_gather(g, b):
        pltpu.make_async_copy(
            lut_hbm.at[idx_v.at[pl.ds(g * CHUNK, CHUNK)]], rows[b],
            gsem[b]).wait()

    def start_out(g, b):
        pltpu.async_copy(
            stage[b], out_hbm.at[pl.ds(base + g * CHUNK, CHUNK)], osem[b])

    def wait_out(g, b):
        pltpu.make_async_copy(
            stage[b], out_hbm.at[pl.ds(base + g * CHUNK, CHUNK)],
            osem[b]).wait()

    start_gather(0, 0)
    start_gather(1, 1)

    @pl.loop(0, NUM_CHUNKS, step=NBUF)
    def _(g):
        for b in range(NBUF):
            gg = g + b
            wait_gather(gg, b)

            @pl.when(gg >= NBUF)
            def _():
                wait_out(gg - NBUF, b)

            _scale_chunk(rows[b], stage[b])

            @pl.when(gg + NBUF < NUM_CHUNKS)
            def _():
                start_gather(gg + NBUF, b)

            start_out(gg, b)

    for b in range(NBUF):
        wait_out(NUM_CHUNKS - NBUF + b, b)


_emb = functools.partial(
    pl.kernel,
    out_type=jax.ShapeDtypeStruct((B_TOTAL, D_MODEL), jnp.float32),
    mesh=plsc.VectorSubcoreMesh(
        core_axis_name="c",
        subcore_axis_name="s",
        num_cores=NUM_CORES,
        num_subcores=NUM_SUBCORES,
    ),
    scratch_types=[
        pltpu.VMEM((ROWS_PER_WORKER,), jnp.int32),
        pltpu.VMEM((CHUNK, D_PAD), jnp.float32),
        pltpu.VMEM((CHUNK, D_PAD), jnp.float32),
        pltpu.VMEM((CHUNK, D_MODEL), jnp.float32),
        pltpu.VMEM((CHUNK, D_MODEL), jnp.float32),
        pltpu.SemaphoreType.DMA,
        pltpu.SemaphoreType.DMA,
        pltpu.SemaphoreType.DMA,
        pltpu.SemaphoreType.DMA,
    ],
    compiler_params=pltpu.CompilerParams(use_tc_tiling_on_sc=True),
)(_emb_body)


@jax.jit
def kernel(x, lut):
    lut_padded = jnp.pad(lut, ((0, 0), (0, D_PAD - D_MODEL)))
    flat = _emb(x.reshape(-1), lut_padded)
    return flat.reshape(x.shape + (D_MODEL,))
